# baseline (speedup 1.0000x reference)
import jax
import jax.numpy as jnp
from jax.experimental import pallas as pl
from jax.experimental.pallas import tpu as pltpu

_D = 128
_NUM_STAGES = 4
_CHUNK = 8192
_N = 65536
_NCHUNK = _N // _CHUNK


def _compute(c_ref, x, R):
    ssq = jnp.sum(x * x, axis=1, keepdims=True)
    inv = jax.lax.rsqrt(ssq)
    nrm = ssq * inv
    xn = x * inv
    xr = jax.lax.dot_general(
        xn, R, (((1,), (0,)), ((), ())),
        preferred_element_type=jnp.float32)
    sign_mask = jnp.int32(-2147483648)
    residual = xr
    for s in range(_NUM_STAGES):
        pos_bits = jax.lax.bitcast_convert_type(c_ref[s, 1], jnp.int32)
        r_bits = jax.lax.bitcast_convert_type(residual, jnp.int32)
        recon = jax.lax.bitcast_convert_type(
            (r_bits & sign_mask) ^ pos_bits, jnp.float32)
        residual = residual - recon
    combined = xr - residual
    out = jax.lax.dot_general(
        combined, R, (((1,), (1,)), ((), ())),
        preferred_element_type=jnp.float32)
    return out * nrm


def _in_copy(x_hbm, xbuf, insem, chunk, slot):
    return pltpu.make_async_copy(
        x_hbm.at[pl.ds(chunk * _CHUNK, _CHUNK), :], xbuf.at[slot],
        insem.at[slot])


def _out_copy(o_hbm, obuf, outsem, chunk, slot):
    return pltpu.make_async_copy(
        obuf.at[slot], o_hbm.at[pl.ds(chunk * _CHUNK, _CHUNK), :],
        outsem.at[slot])


def _msrc_kernel(c_ref, x_hbm, r_ref, o_hbm, xbuf, obuf, insem, outsem):
    i = pl.program_id(0)
    slot = jax.lax.rem(i, 2)
    nslot = jax.lax.rem(i + 1, 2)

    @pl.when(i == 0)
    def _prologue():
        _in_copy(x_hbm, xbuf, insem, 0, 0).start()

    @pl.when(i + 1 < _NCHUNK)
    def _prefetch():
        _in_copy(x_hbm, xbuf, insem, i + 1, nslot).start()

    @pl.when(i >= 2)
    def _reclaim():
        _out_copy(o_hbm, obuf, outsem, i - 2, slot).wait()

    _in_copy(x_hbm, xbuf, insem, i, slot).wait()
    obuf[slot] = _compute(c_ref, xbuf[slot], r_ref[...])
    _out_copy(o_hbm, obuf, outsem, i, slot).start()

    @pl.when(i == _NCHUNK - 1)
    def _epilogue():
        _out_copy(o_hbm, obuf, outsem, i - 1, nslot).wait()
        _out_copy(o_hbm, obuf, outsem, i, slot).wait()


def kernel(x, R, centroids):
    n, d = x.shape
    return pl.pallas_call(
        _msrc_kernel,
        grid=(_NCHUNK,),
        in_specs=[
            pl.BlockSpec(memory_space=pltpu.SMEM),
            pl.BlockSpec(memory_space=pltpu.MemorySpace.HBM),
            pl.BlockSpec((d, d), lambda i: (0, 0)),
        ],
        out_specs=pl.BlockSpec(memory_space=pltpu.MemorySpace.HBM),
        out_shape=jax.ShapeDtypeStruct((n, d), jnp.float32),
        scratch_shapes=[
            pltpu.VMEM((2, _CHUNK, _D), jnp.float32),
            pltpu.VMEM((2, _CHUNK, _D), jnp.float32),
            pltpu.SemaphoreType.DMA((2,)),
            pltpu.SemaphoreType.DMA((2,)),
        ],
        compiler_params=pltpu.CompilerParams(
            dimension_semantics=("arbitrary",)),
    )(centroids, x, R)


# speedup vs baseline: 1.0158x; 1.0158x over previous
import jax
import jax.numpy as jnp
from jax.experimental import pallas as pl
from jax.experimental.pallas import tpu as pltpu

_D = 128
_NUM_STAGES = 4
_CHUNK = 16384
_N = 65536
_NCHUNK = _N // _CHUNK


def _compute(c_ref, x, R):
    ssq = jnp.sum(x * x, axis=1, keepdims=True)
    inv = jax.lax.rsqrt(ssq)
    nrm = ssq * inv
    xn = x * inv
    xr = jax.lax.dot_general(
        xn, R, (((1,), (0,)), ((), ())),
        preferred_element_type=jnp.float32)
    sign_mask = jnp.int32(-2147483648)
    residual = xr
    for s in range(_NUM_STAGES):
        pos_bits = jax.lax.bitcast_convert_type(c_ref[s, 1], jnp.int32)
        r_bits = jax.lax.bitcast_convert_type(residual, jnp.int32)
        recon = jax.lax.bitcast_convert_type(
            (r_bits & sign_mask) ^ pos_bits, jnp.float32)
        residual = residual - recon
    combined = xr - residual
    out = jax.lax.dot_general(
        combined, R, (((1,), (1,)), ((), ())),
        preferred_element_type=jnp.float32)
    return out * nrm


def _in_copy(x_hbm, xbuf, insem, chunk, slot):
    return pltpu.make_async_copy(
        x_hbm.at[pl.ds(chunk * _CHUNK, _CHUNK), :], xbuf.at[slot],
        insem.at[slot])


def _out_copy(o_hbm, obuf, outsem, chunk, slot):
    return pltpu.make_async_copy(
        obuf.at[slot], o_hbm.at[pl.ds(chunk * _CHUNK, _CHUNK), :],
        outsem.at[slot])


def _msrc_kernel(c_ref, x_hbm, r_ref, o_hbm, xbuf, obuf, insem, outsem):
    i = pl.program_id(0)
    slot = jax.lax.rem(i, 2)
    nslot = jax.lax.rem(i + 1, 2)

    @pl.when(i == 0)
    def _prologue():
        _in_copy(x_hbm, xbuf, insem, 0, 0).start()

    @pl.when(i + 1 < _NCHUNK)
    def _prefetch():
        _in_copy(x_hbm, xbuf, insem, i + 1, nslot).start()

    @pl.when(i >= 2)
    def _reclaim():
        _out_copy(o_hbm, obuf, outsem, i - 2, slot).wait()

    _in_copy(x_hbm, xbuf, insem, i, slot).wait()
    obuf[slot] = _compute(c_ref, xbuf[slot], r_ref[...])
    _out_copy(o_hbm, obuf, outsem, i, slot).start()

    @pl.when(i == _NCHUNK - 1)
    def _epilogue():
        _out_copy(o_hbm, obuf, outsem, i - 1, nslot).wait()
        _out_copy(o_hbm, obuf, outsem, i, slot).wait()


def kernel(x, R, centroids):
    n, d = x.shape
    return pl.pallas_call(
        _msrc_kernel,
        grid=(_NCHUNK,),
        in_specs=[
            pl.BlockSpec(memory_space=pltpu.SMEM),
            pl.BlockSpec(memory_space=pltpu.MemorySpace.HBM),
            pl.BlockSpec((d, d), lambda i: (0, 0)),
        ],
        out_specs=pl.BlockSpec(memory_space=pltpu.MemorySpace.HBM),
        out_shape=jax.ShapeDtypeStruct((n, d), jnp.float32),
        scratch_shapes=[
            pltpu.VMEM((2, _CHUNK, _D), jnp.float32),
            pltpu.VMEM((2, _CHUNK, _D), jnp.float32),
            pltpu.SemaphoreType.DMA((2,)),
            pltpu.SemaphoreType.DMA((2,)),
        ],
        compiler_params=pltpu.CompilerParams(
            dimension_semantics=("arbitrary",)),
    )(centroids, x, R)


# speedup vs baseline: 1.0469x; 1.0307x over previous
import jax
import jax.numpy as jnp
from jax.experimental import pallas as pl
from jax.experimental.pallas import tpu as pltpu

_D = 128
_NUM_STAGES = 4
_BLOCK = 16384


def _msrc_kernel(c_ref, x_ref, r_ref, o_ref):
    x = x_ref[...]
    R = r_ref[...]
    ssq = jnp.sum(x * x, axis=1, keepdims=True)
    inv = jax.lax.rsqrt(ssq)
    nrm = ssq * inv
    xn = x * inv
    xr = jax.lax.dot_general(
        xn, R, (((1,), (0,)), ((), ())),
        preferred_element_type=jnp.float32)
    sign_mask = jnp.int32(-2147483648)
    residual = xr
    for s in range(_NUM_STAGES):
        pos_bits = jax.lax.bitcast_convert_type(c_ref[s, 1], jnp.int32)
        r_bits = jax.lax.bitcast_convert_type(residual, jnp.int32)
        recon = jax.lax.bitcast_convert_type(
            (r_bits & sign_mask) ^ pos_bits, jnp.float32)
        residual = residual - recon
    combined = xr - residual
    out = jax.lax.dot_general(
        combined, R, (((1,), (1,)), ((), ())), preferred_element_type=jnp.float32)
    o_ref[...] = out * nrm


def kernel(x, R, centroids):
    n, d = x.shape
    grid = (n // _BLOCK,)
    return pl.pallas_call(
        _msrc_kernel,
        grid=grid,
        in_specs=[
            pl.BlockSpec(memory_space=pltpu.SMEM),
            pl.BlockSpec((_BLOCK, d), lambda i: (i, 0)),
            pl.BlockSpec((d, d), lambda i: (0, 0)),
        ],
        out_specs=pl.BlockSpec((_BLOCK, d), lambda i: (i, 0)),
        out_shape=jax.ShapeDtypeStruct((n, d), jnp.float32),
        compiler_params=pltpu.CompilerParams(
            dimension_semantics=("arbitrary",)),
    )(centroids, x, R)
